# baseline (speedup 1.0000x reference)
import jax
import jax.numpy as jnp
from jax import lax
from jax.experimental import pallas as pl
from jax.experimental.pallas import tpu as pltpu

_EPS = 1e-09
_CHUNK = 8192


def _make_pair_counts_kernel(nchunk, half_rows):
    def _pair_counts_kernel(z_hbm, zt_hbm, out_ref, zbuf, ztbuf, zsems, ztsems):
        h = pl.program_id(0)
        base = h * half_rows

        copies = []
        for i in range(nchunk):
            cz = pltpu.make_async_copy(
                z_hbm.at[pl.ds(base + i * _CHUNK, _CHUNK), :],
                zbuf.at[pl.ds(i * _CHUNK, _CHUNK), :],
                zsems.at[i])
            czt = pltpu.make_async_copy(
                zt_hbm.at[pl.ds(base + i * _CHUNK, _CHUNK), :],
                ztbuf.at[pl.ds(i * _CHUNK, _CHUNK), :],
                ztsems.at[i])
            cz.start()
            czt.start()
            copies.append((cz, czt))

        c = out_ref.shape[1]
        acc = jnp.zeros((c, c), jnp.float32)
        for i, (cz, czt) in enumerate(copies):
            cz.wait()
            czt.wait()
            zb = zbuf[pl.ds(i * _CHUNK, _CHUNK), :].astype(jnp.bfloat16)
            ztb = ztbuf[pl.ds(i * _CHUNK, _CHUNK), :].astype(jnp.bfloat16)
            acc = acc + lax.dot_general(
                zb, ztb,
                dimension_numbers=(((0,), (0,)), ((), ())),
                preferred_element_type=jnp.float32,
            )
        out_ref[0] = acc

    return _pair_counts_kernel


def _loss_kernel(parts_ref, loss_ref):
    P = parts_ref[0] + parts_ref[1]
    P = (P + P.T) * (0.5 / jnp.sum(P))
    P = jnp.maximum(P, _EPS)
    Pi = jnp.sum(P, axis=1, keepdims=True)
    Pj = jnp.sum(P, axis=0, keepdims=True)
    loss_ref[0, 0] = (jnp.sum(Pi * jnp.log(Pi))
                      + jnp.sum(Pj * jnp.log(Pj))
                      - jnp.sum(P * jnp.log(P)))


def kernel(z, zt):
    n, c = z.shape
    assert zt.shape == (n, c)

    span = 2 * _CHUNK
    n_pad = -(-n // span) * span
    if n_pad != n:
        pad = n_pad - n
        z = jnp.pad(z, ((0, pad), (0, 0)))
        zt = jnp.pad(zt, ((0, pad), (0, 0)))
    half_rows = n_pad // 2
    nchunk = half_rows // _CHUNK

    scratch_bytes = 2 * half_rows * c * 4

    partials = pl.pallas_call(
        _make_pair_counts_kernel(nchunk, half_rows),
        out_shape=jax.ShapeDtypeStruct((2, c, c), jnp.float32),
        grid=(2,),
        in_specs=[
            pl.BlockSpec(memory_space=pl.ANY),
            pl.BlockSpec(memory_space=pl.ANY),
        ],
        out_specs=pl.BlockSpec((1, c, c), lambda h: (h, 0, 0)),
        scratch_shapes=[
            pltpu.VMEM((half_rows, c), jnp.float32),
            pltpu.VMEM((half_rows, c), jnp.float32),
            pltpu.SemaphoreType.DMA((nchunk,)),
            pltpu.SemaphoreType.DMA((nchunk,)),
        ],
        compiler_params=pltpu.CompilerParams(
            dimension_semantics=("parallel",),
            vmem_limit_bytes=min(scratch_bytes + 16 * 1024 * 1024,
                                 56 * 1024 * 1024),
        ),
        cost_estimate=pl.CostEstimate(
            flops=2 * n_pad * c * c,
            transcendentals=0,
            bytes_accessed=2 * n_pad * c * 4 + 2 * c * c * 4,
        ),
    )(z, zt)

    loss = pl.pallas_call(
        _loss_kernel,
        out_shape=jax.ShapeDtypeStruct((1, 1), jnp.float32),
        in_specs=[pl.BlockSpec((2, c, c), lambda: (0, 0, 0))],
        out_specs=pl.BlockSpec(memory_space=pltpu.MemorySpace.SMEM),
        cost_estimate=pl.CostEstimate(
            flops=8 * c * c,
            transcendentals=c * c + 2 * c,
            bytes_accessed=2 * c * c * 4 + 4,
        ),
    )(partials)
    return loss[0, 0]


# speedup vs baseline: 1.1553x; 1.1553x over previous
import jax
import jax.numpy as jnp
from jax import lax
from jax.experimental import pallas as pl
from jax.experimental.pallas import tpu as pltpu

_EPS = 1e-09


def _fused_kernel(z_ref, zt_ref, loss_ref, acc_ref):
    k = pl.program_id(0)

    @pl.when(k == 0)
    def _zero():
        acc_ref[...] = jnp.zeros_like(acc_ref)

    zb = z_ref[...].astype(jnp.bfloat16)
    ztb = zt_ref[...].astype(jnp.bfloat16)
    acc_ref[...] += lax.dot_general(
        zb, ztb,
        dimension_numbers=(((0,), (0,)), ((), ())),
        preferred_element_type=jnp.float32,
    )

    @pl.when(k == pl.num_programs(0) - 1)
    def _epilogue():
        P = acc_ref[...]
        P = (P + P.T) * (0.5 / jnp.sum(P))
        P = jnp.maximum(P, _EPS)
        Pi = jnp.sum(P, axis=1, keepdims=True)
        Pj = jnp.sum(P, axis=0, keepdims=True)
        loss_ref[0, 0] = (jnp.sum(Pi * jnp.log(Pi))
                          + jnp.sum(Pj * jnp.log(Pj))
                          - jnp.sum(P * jnp.log(P)))


def kernel(z, zt):
    n, c = z.shape
    assert zt.shape == (n, c)

    tile_n = 8192
    n_pad = -(-n // tile_n) * tile_n
    if n_pad != n:
        pad = n_pad - n
        z = jnp.pad(z, ((0, pad), (0, 0)))
        zt = jnp.pad(zt, ((0, pad), (0, 0)))
    kt = n_pad // tile_n

    loss = pl.pallas_call(
        _fused_kernel,
        out_shape=jax.ShapeDtypeStruct((1, 1), jnp.float32),
        grid=(kt,),
        in_specs=[
            pl.BlockSpec((tile_n, c), lambda k: (k, 0)),
            pl.BlockSpec((tile_n, c), lambda k: (k, 0)),
        ],
        out_specs=pl.BlockSpec(memory_space=pltpu.MemorySpace.SMEM),
        scratch_shapes=[pltpu.VMEM((c, c), jnp.float32)],
        compiler_params=pltpu.CompilerParams(
            dimension_semantics=("arbitrary",),
            vmem_limit_bytes=56 * 1024 * 1024,
        ),
        cost_estimate=pl.CostEstimate(
            flops=2 * n_pad * c * c,
            transcendentals=c * c + 2 * c,
            bytes_accessed=2 * n_pad * c * 4 + 4,
        ),
    )(z, zt)
    return loss[0, 0]
